# baseline (speedup 1.0000x reference)
import jax
import jax.numpy as jnp
from jax.experimental import pallas as pl
from jax.experimental.pallas import tpu as pltpu

_C1 = 0.01 ** 2
_C2 = 0.03 ** 2

_S = 3072
_BLK = 512
_DN = 192 * 256
_MED_K = (_DN - 1) // 2


def _box3(a):
    h_, w_ = a.shape
    left = jnp.concatenate([a[:, 1:2], a[:, : w_ - 1]], axis=1)
    right = jnp.concatenate([a[:, 1:w_], a[:, w_ - 2 : w_ - 1]], axis=1)
    h = left + a + right
    up = jnp.concatenate([h[1:2, :], h[: h_ - 1, :]], axis=0)
    dn = jnp.concatenate([h[1:h_, :], h[h_ - 2 : h_ - 1, :]], axis=0)
    return up + h + dn


def _sortable(x):
    b = jax.lax.bitcast_convert_type(x, jnp.int32)
    return jnp.where(b < 0, ~b, b ^ jnp.int32(-(2 ** 31)))


def _unsortable(prefix):
    fb = jnp.where(prefix < 0, prefix ^ jnp.int32(-(2 ** 31)), ~prefix)
    return jax.lax.bitcast_convert_type(fb, jnp.float32)


def _median_lower2(x, y):
    ux = _sortable(x)
    uy = _sortable(y)

    def body(i, carry):
        px, rkx, py, rky, done = carry
        bit = jnp.int32(31) - i
        bmask = jnp.left_shift(jnp.int32(1), bit)
        m = done | bmask
        cx = jnp.sum(((ux & m) == px).astype(jnp.int32))
        cy = jnp.sum(((uy & m) == py).astype(jnp.int32))
        tx = rkx >= cx
        ty = rky >= cy
        px = jnp.where(tx, px | bmask, px)
        rkx = jnp.where(tx, rkx - cx, rkx)
        py = jnp.where(ty, py | bmask, py)
        rky = jnp.where(ty, rky - cy, rky)
        return px, rkx, py, rky, done | bmask

    k = jnp.int32(_MED_K)
    px, _, py, _, _ = jax.lax.fori_loop(
        0, 32, body, (jnp.int32(0), k, jnp.int32(0), k, jnp.int32(0)))
    return _unsortable(px), _unsortable(py)


def _loss_body(rgbp_ref, rgbg_ref, dp_ref, dg_ref, t_ref, x_ref, yt_ref,
               im1_ref, im2_ref, m_ref, d1p_ref, d2_ref, d2p_ref, d1_ref,
               out_ref):
    f32 = jnp.float32

    dr = rgbp_ref[...] - rgbg_ref[...]
    rgb_full = jnp.sum(dr * dr) * f32(1.0 / 2048.0)

    pred = dp_ref[...]
    gt = dg_ref[...]
    tp, tg = _median_lower2(pred, gt)
    sp = jnp.mean(jnp.abs(pred - tp))
    sg = jnp.mean(jnp.abs(gt - tg))
    pn = (pred - tp) / sp
    gn = (gt - tg) / sg
    depth_loss = jnp.mean((pn - gn) ** 2)

    t = t_ref[...]
    td = t[1:, :] - t[:-1, :]
    tn = jnp.sqrt(jnp.sum(td * td, axis=1, keepdims=True))
    loss_d1 = jnp.sum(tn) * f32(1.0 / 59.0)
    dd = tn[1:, :] - tn[:-1, :]
    loss_d2 = jnp.sum(dd * dd) * f32(1.0 / 58.0)

    ym = yt_ref[...].T
    y0 = ym[0:1, :]
    y1 = ym[1:2, :]
    y2 = ym[2:3, :]
    y2row = y0 * y0 + y1 * y1 + y2 * y2

    def blk(i, carry):
        racc, colmin = carry
        xb = x_ref[pl.ds(i * _BLK, _BLK), :]
        x0 = xb[:, 0:1]
        x1 = xb[:, 1:2]
        x2 = xb[:, 2:3]
        x2col = x0 * x0 + x1 * x1 + x2 * x2
        base = x2col + y2row
        cross = x0 * y0 + x1 * y1 + x2 * y2
        dsq = base - 2.0 * cross
        rmin = jnp.maximum(jnp.min(dsq, axis=1, keepdims=True), f32(0.0))
        racc = racc + jnp.sum(jnp.sqrt(rmin))
        colmin = jnp.minimum(colmin, jnp.min(dsq, axis=0, keepdims=True))
        return racc, colmin

    racc, colmin = jax.lax.fori_loop(
        0, _S // _BLK, blk,
        (f32(0.0), jnp.full((1, _S), jnp.inf, f32)))
    colmin = jnp.maximum(colmin, f32(0.0))
    pc_loss = racc * f32(1.0 / _S) + jnp.sum(jnp.sqrt(colmin)) * f32(1.0 / _S)

    m = m_ref[...].astype(f32)
    msum = jnp.sum(m) * f32(3.0)
    acc = f32(0.0)
    for c in range(3):
        x = im1_ref[c]
        y = im2_ref[c]
        mu_x = _box3(x) * f32(1.0 / 9.0)
        mu_y = _box3(y) * f32(1.0 / 9.0)
        sxx = _box3(x * x) * f32(1.0 / 9.0) - mu_x * mu_x
        syy = _box3(y * y) * f32(1.0 / 9.0) - mu_y * mu_y
        sxy = _box3(x * y) * f32(1.0 / 9.0) - mu_x * mu_y
        n = (2.0 * mu_x * mu_y + _C1) * (2.0 * sxy + _C2)
        d = (mu_x * mu_x + mu_y * mu_y + _C1) * (sxx + syy + _C2)
        smap = jnp.clip((1.0 - n / d) * 0.5, 0.0, 1.0)
        diff = 0.15 * jnp.clip(jnp.abs(x - y), 0.0, 1.0) + 0.85 * smap
        acc = acc + jnp.sum(diff * m)
    rgb_s = acc / msum

    dca = jnp.sum(jnp.abs(d1p_ref[...] - d2_ref[...])) * f32(1.0 / 65536.0)
    dcb = jnp.sum(jnp.abs(d2p_ref[...] - d1_ref[...])) * f32(1.0 / 65536.0)
    dc = 0.5 * dca + 0.5 * dcb

    loss = (rgb_full + 0.04 * depth_loss + 0.1 * loss_d1 + 0.1 * loss_d2
            + pc_loss + rgb_s + dc)

    out_ref[0] = loss
    out_ref[1] = rgb_full
    out_ref[2] = depth_loss
    out_ref[3] = loss_d1
    out_ref[4] = loss_d2
    out_ref[5] = pc_loss
    out_ref[6] = rgb_s
    out_ref[7] = dc


def kernel(rgb_pred, rgb_gt, depth_pred, depth_gt, t_list, X, Y, rgb_pc1,
           rgb_pc1_proj, valid_points, d1_proj, d2, d2_proj, d1):
    xm = X[0]
    yt = Y[0]
    im1 = rgb_pc1[0]
    im2 = rgb_pc1_proj[0]
    mask = valid_points[0, 0]
    out = pl.pallas_call(
        _loss_body,
        out_shape=jax.ShapeDtypeStruct((8,), jnp.float32),
        out_specs=pl.BlockSpec(memory_space=pltpu.SMEM),
    )(rgb_pred[0], rgb_gt[0], depth_pred, depth_gt, t_list, xm, yt,
      im1, im2, mask,
      d1_proj.reshape(512, 128), d2.reshape(512, 128),
      d2_proj.reshape(512, 128), d1.reshape(512, 128))
    return (out[0], out[1], out[2], out[3], out[4], out[5], out[6], out[7])


# speedup vs baseline: 1.0213x; 1.0213x over previous
import jax
import jax.numpy as jnp
from jax.experimental import pallas as pl
from jax.experimental.pallas import tpu as pltpu

_C1 = 0.01 ** 2
_C2 = 0.03 ** 2

_S = 3072
_BLK = 512
_DN = 192 * 256
_MED_K = (_DN - 1) // 2


def _box3(a):
    h_, w_ = a.shape
    left = jnp.concatenate([a[:, 1:2], a[:, : w_ - 1]], axis=1)
    right = jnp.concatenate([a[:, 1:w_], a[:, w_ - 2 : w_ - 1]], axis=1)
    h = left + a + right
    up = jnp.concatenate([h[1:2, :], h[: h_ - 1, :]], axis=0)
    dn = jnp.concatenate([h[1:h_, :], h[h_ - 2 : h_ - 1, :]], axis=0)
    return up + h + dn


def _sortable(x):
    b = jax.lax.bitcast_convert_type(x, jnp.int32)
    return jnp.where(b < 0, ~b, b ^ jnp.int32(-(2 ** 31)))


def _unsortable(prefix):
    fb = jnp.where(prefix < 0, prefix ^ jnp.int32(-(2 ** 31)), ~prefix)
    return jax.lax.bitcast_convert_type(fb, jnp.float32)


def _median_lower2(x, y):
    ux = _sortable(x)
    uy = _sortable(y)

    def body(i, carry):
        px, rkx, py, rky, done = carry
        bit = jnp.int32(31) - i
        bmask = jnp.left_shift(jnp.int32(1), bit)
        m = done | bmask
        cx = jnp.sum(((ux & m) == px).astype(jnp.int32))
        cy = jnp.sum(((uy & m) == py).astype(jnp.int32))
        tx = rkx >= cx
        ty = rky >= cy
        px = jnp.where(tx, px | bmask, px)
        rkx = jnp.where(tx, rkx - cx, rkx)
        py = jnp.where(ty, py | bmask, py)
        rky = jnp.where(ty, rky - cy, rky)
        return px, rkx, py, rky, done | bmask

    k = jnp.int32(_MED_K)
    px, _, py, _, _ = jax.lax.fori_loop(
        0, 32, body, (jnp.int32(0), k, jnp.int32(0), k, jnp.int32(0)))
    return _unsortable(px), _unsortable(py)


def _loss_body(rgbp_ref, rgbg_ref, dp_ref, dg_ref, t_ref, x_ref, yt_ref,
               im1_hbm, im2_hbm, m_ref, d1p_hbm, d2_hbm, d2p_hbm, d1_hbm,
               out_ref,
               im1_ref, im2_ref, d1p_ref, d2_ref, d2p_ref, d1_ref, sems):
    f32 = jnp.float32

    copies = [
        pltpu.make_async_copy(im1_hbm, im1_ref, sems.at[0]),
        pltpu.make_async_copy(im2_hbm, im2_ref, sems.at[1]),
        pltpu.make_async_copy(d1p_hbm, d1p_ref, sems.at[2]),
        pltpu.make_async_copy(d2_hbm, d2_ref, sems.at[3]),
        pltpu.make_async_copy(d2p_hbm, d2p_ref, sems.at[4]),
        pltpu.make_async_copy(d1_hbm, d1_ref, sems.at[5]),
    ]
    for c in copies:
        c.start()

    ym = yt_ref[...].T
    y0 = ym[0:1, :]
    y1 = ym[1:2, :]
    y2 = ym[2:3, :]
    y2row = y0 * y0 + y1 * y1 + y2 * y2

    def blk(i, carry):
        racc, colmin = carry
        xb = x_ref[pl.ds(i * _BLK, _BLK), :]
        x0 = xb[:, 0:1]
        x1 = xb[:, 1:2]
        x2 = xb[:, 2:3]
        x2col = x0 * x0 + x1 * x1 + x2 * x2
        base = x2col + y2row
        cross = x0 * y0 + x1 * y1 + x2 * y2
        dsq = base - 2.0 * cross
        rmin = jnp.maximum(jnp.min(dsq, axis=1, keepdims=True), f32(0.0))
        racc = racc + jnp.sum(jnp.sqrt(rmin))
        colmin = jnp.minimum(colmin, jnp.min(dsq, axis=0, keepdims=True))
        return racc, colmin

    racc, colmin = jax.lax.fori_loop(
        0, _S // _BLK, blk,
        (f32(0.0), jnp.full((1, _S), jnp.inf, f32)))
    colmin = jnp.maximum(colmin, f32(0.0))
    pc_loss = racc * f32(1.0 / _S) + jnp.sum(jnp.sqrt(colmin)) * f32(1.0 / _S)

    dr = rgbp_ref[...] - rgbg_ref[...]
    rgb_full = jnp.sum(dr * dr) * f32(1.0 / 2048.0)

    pred = dp_ref[...]
    gt = dg_ref[...]
    tp, tg = _median_lower2(pred, gt)
    sp = jnp.mean(jnp.abs(pred - tp))
    sg = jnp.mean(jnp.abs(gt - tg))
    pn = (pred - tp) / sp
    gn = (gt - tg) / sg
    depth_loss = jnp.mean((pn - gn) ** 2)

    t = t_ref[...]
    td = t[1:, :] - t[:-1, :]
    tn = jnp.sqrt(jnp.sum(td * td, axis=1, keepdims=True))
    loss_d1 = jnp.sum(tn) * f32(1.0 / 59.0)
    dd = tn[1:, :] - tn[:-1, :]
    loss_d2 = jnp.sum(dd * dd) * f32(1.0 / 58.0)

    copies[2].wait()
    copies[3].wait()
    copies[4].wait()
    copies[5].wait()
    dca = jnp.sum(jnp.abs(d1p_ref[...] - d2_ref[...])) * f32(1.0 / 65536.0)
    dcb = jnp.sum(jnp.abs(d2p_ref[...] - d1_ref[...])) * f32(1.0 / 65536.0)
    dc = 0.5 * dca + 0.5 * dcb

    copies[0].wait()
    copies[1].wait()
    m = m_ref[...].astype(f32)
    msum = jnp.sum(m) * f32(3.0)
    acc = f32(0.0)
    for c in range(3):
        x = im1_ref[c]
        y = im2_ref[c]
        mu_x = _box3(x) * f32(1.0 / 9.0)
        mu_y = _box3(y) * f32(1.0 / 9.0)
        sxx = _box3(x * x) * f32(1.0 / 9.0) - mu_x * mu_x
        syy = _box3(y * y) * f32(1.0 / 9.0) - mu_y * mu_y
        sxy = _box3(x * y) * f32(1.0 / 9.0) - mu_x * mu_y
        n = (2.0 * mu_x * mu_y + _C1) * (2.0 * sxy + _C2)
        d = (mu_x * mu_x + mu_y * mu_y + _C1) * (sxx + syy + _C2)
        smap = jnp.clip((1.0 - n / d) * 0.5, 0.0, 1.0)
        diff = 0.15 * jnp.clip(jnp.abs(x - y), 0.0, 1.0) + 0.85 * smap
        acc = acc + jnp.sum(diff * m)
    rgb_s = acc / msum

    loss = (rgb_full + 0.04 * depth_loss + 0.1 * loss_d1 + 0.1 * loss_d2
            + pc_loss + rgb_s + dc)

    out_ref[0] = loss
    out_ref[1] = rgb_full
    out_ref[2] = depth_loss
    out_ref[3] = loss_d1
    out_ref[4] = loss_d2
    out_ref[5] = pc_loss
    out_ref[6] = rgb_s
    out_ref[7] = dc


def kernel(rgb_pred, rgb_gt, depth_pred, depth_gt, t_list, X, Y, rgb_pc1,
           rgb_pc1_proj, valid_points, d1_proj, d2, d2_proj, d1):
    f32 = jnp.float32
    xm = X[0]
    yt = Y[0]
    im1 = rgb_pc1[0]
    im2 = rgb_pc1_proj[0]
    mask = valid_points[0, 0]
    vmem = pl.BlockSpec(memory_space=pltpu.VMEM)
    hbm = pl.BlockSpec(memory_space=pltpu.MemorySpace.HBM)
    out = pl.pallas_call(
        _loss_body,
        out_shape=jax.ShapeDtypeStruct((8,), jnp.float32),
        in_specs=[vmem, vmem, vmem, vmem, vmem, vmem, vmem,
                  hbm, hbm, vmem, hbm, hbm, hbm, hbm],
        out_specs=pl.BlockSpec(memory_space=pltpu.SMEM),
        scratch_shapes=[
            pltpu.VMEM((3, 384, 384), f32), pltpu.VMEM((3, 384, 384), f32),
            pltpu.VMEM((512, 128), f32), pltpu.VMEM((512, 128), f32),
            pltpu.VMEM((512, 128), f32), pltpu.VMEM((512, 128), f32),
            pltpu.SemaphoreType.DMA((6,)),
        ],
    )(rgb_pred[0], rgb_gt[0], depth_pred, depth_gt, t_list, xm, yt,
      im1, im2, mask,
      d1_proj.reshape(512, 128), d2.reshape(512, 128),
      d2_proj.reshape(512, 128), d1.reshape(512, 128))
    return (out[0], out[1], out[2], out[3], out[4], out[5], out[6], out[7])


# speedup vs baseline: 1.0541x; 1.0321x over previous
import jax
import jax.numpy as jnp
from jax.experimental import pallas as pl
from jax.experimental.pallas import tpu as pltpu

_C1 = 0.01 ** 2
_C2 = 0.03 ** 2

_S = 3072
_BLK = 512
_DN = 192 * 256
_MED_K = (_DN - 1) // 2


def _box3(a):
    h_, w_ = a.shape
    left = jnp.concatenate([a[:, 1:2], a[:, : w_ - 1]], axis=1)
    right = jnp.concatenate([a[:, 1:w_], a[:, w_ - 2 : w_ - 1]], axis=1)
    h = left + a + right
    up = jnp.concatenate([h[1:2, :], h[: h_ - 1, :]], axis=0)
    dn = jnp.concatenate([h[1:h_, :], h[h_ - 2 : h_ - 1, :]], axis=0)
    return up + h + dn


def _sortable(x):
    b = jax.lax.bitcast_convert_type(x, jnp.int32)
    return jnp.where(b < 0, ~b, b ^ jnp.int32(-(2 ** 31)))


def _unsortable(prefix):
    fb = jnp.where(prefix < 0, prefix ^ jnp.int32(-(2 ** 31)), ~prefix)
    return jax.lax.bitcast_convert_type(fb, jnp.float32)


def _median_lower2(x, y):
    ux = _sortable(x)
    uy = _sortable(y)

    def body(i, carry):
        px, rkx, py, rky, done = carry
        bit = jnp.int32(31) - i
        bmask = jnp.left_shift(jnp.int32(1), bit)
        m = done | bmask
        cx = jnp.sum(((ux & m) == px).astype(jnp.int32))
        cy = jnp.sum(((uy & m) == py).astype(jnp.int32))
        tx = rkx >= cx
        ty = rky >= cy
        px = jnp.where(tx, px | bmask, px)
        rkx = jnp.where(tx, rkx - cx, rkx)
        py = jnp.where(ty, py | bmask, py)
        rky = jnp.where(ty, rky - cy, rky)
        return px, rkx, py, rky, done | bmask

    k = jnp.int32(_MED_K)
    px, _, py, _, _ = jax.lax.fori_loop(
        0, 32, body, (jnp.int32(0), k, jnp.int32(0), k, jnp.int32(0)))
    return _unsortable(px), _unsortable(py)


def _loss_body(rgbp_ref, rgbg_ref, dp_ref, dg_ref, t_ref, x_ref, y_ref,
               im1_hbm, im2_hbm, m_ref, d1p_hbm, d2_hbm, d2p_hbm, d1_hbm,
               out_ref,
               im1_ref, im2_ref, d1p_ref, d2_ref, d2p_ref, d1_ref, sems):
    f32 = jnp.float32

    copies = [
        pltpu.make_async_copy(im1_hbm.at[0], im1_ref, sems.at[0]),
        pltpu.make_async_copy(im2_hbm.at[0], im2_ref, sems.at[1]),
        pltpu.make_async_copy(d1p_hbm.reshape(512, 128), d1p_ref, sems.at[2]),
        pltpu.make_async_copy(d2_hbm.reshape(512, 128), d2_ref, sems.at[3]),
        pltpu.make_async_copy(d2p_hbm.reshape(512, 128), d2p_ref, sems.at[4]),
        pltpu.make_async_copy(d1_hbm.reshape(512, 128), d1_ref, sems.at[5]),
    ]
    for c in copies:
        c.start()

    ym = y_ref[0].T
    y0 = ym[0:1, :]
    y1 = ym[1:2, :]
    y2 = ym[2:3, :]

    def blk(i, carry):
        racc, colmin = carry
        xb = x_ref[0, pl.ds(i * _BLK, _BLK), :]
        e0 = xb[:, 0:1] - y0
        e1 = xb[:, 1:2] - y1
        e2 = xb[:, 2:3] - y2
        dsq = e0 * e0 + e1 * e1 + e2 * e2
        racc = racc + jnp.sum(jnp.sqrt(jnp.min(dsq, axis=1, keepdims=True)))
        colmin = jnp.minimum(colmin, jnp.min(dsq, axis=0, keepdims=True))
        return racc, colmin

    racc, colmin = jax.lax.fori_loop(
        0, _S // _BLK, blk,
        (f32(0.0), jnp.full((1, _S), jnp.inf, f32)))
    pc_loss = racc * f32(1.0 / _S) + jnp.sum(jnp.sqrt(colmin)) * f32(1.0 / _S)

    dr = rgbp_ref[0] - rgbg_ref[0]
    rgb_full = jnp.sum(dr * dr) * f32(1.0 / 2048.0)

    pred = dp_ref[...]
    gt = dg_ref[...]
    tp, tg = _median_lower2(pred, gt)
    sp = jnp.mean(jnp.abs(pred - tp))
    sg = jnp.mean(jnp.abs(gt - tg))
    pn = (pred - tp) / sp
    gn = (gt - tg) / sg
    depth_loss = jnp.mean((pn - gn) ** 2)

    t = t_ref[...]
    td = t[1:, :] - t[:-1, :]
    tn = jnp.sqrt(jnp.sum(td * td, axis=1, keepdims=True))
    loss_d1 = jnp.sum(tn) * f32(1.0 / 59.0)
    dd = tn[1:, :] - tn[:-1, :]
    loss_d2 = jnp.sum(dd * dd) * f32(1.0 / 58.0)

    copies[2].wait()
    copies[3].wait()
    copies[4].wait()
    copies[5].wait()
    dca = jnp.sum(jnp.abs(d1p_ref[...] - d2_ref[...])) * f32(1.0 / 65536.0)
    dcb = jnp.sum(jnp.abs(d2p_ref[...] - d1_ref[...])) * f32(1.0 / 65536.0)
    dc = 0.5 * dca + 0.5 * dcb

    copies[0].wait()
    copies[1].wait()
    m = m_ref[0, 0].astype(f32)
    msum = jnp.sum(m) * f32(3.0)
    acc = f32(0.0)
    for c in range(3):
        x = im1_ref[c]
        y = im2_ref[c]
        mu_x = _box3(x) * f32(1.0 / 9.0)
        mu_y = _box3(y) * f32(1.0 / 9.0)
        sxx = _box3(x * x) * f32(1.0 / 9.0) - mu_x * mu_x
        syy = _box3(y * y) * f32(1.0 / 9.0) - mu_y * mu_y
        sxy = _box3(x * y) * f32(1.0 / 9.0) - mu_x * mu_y
        n = (2.0 * mu_x * mu_y + _C1) * (2.0 * sxy + _C2)
        d = (mu_x * mu_x + mu_y * mu_y + _C1) * (sxx + syy + _C2)
        smap = jnp.clip((1.0 - n / d) * 0.5, 0.0, 1.0)
        diff = 0.15 * jnp.clip(jnp.abs(x - y), 0.0, 1.0) + 0.85 * smap
        acc = acc + jnp.sum(diff * m)
    rgb_s = acc / msum

    loss = (rgb_full + 0.04 * depth_loss + 0.1 * loss_d1 + 0.1 * loss_d2
            + pc_loss + rgb_s + dc)

    out_ref[0] = loss
    out_ref[1] = rgb_full
    out_ref[2] = depth_loss
    out_ref[3] = loss_d1
    out_ref[4] = loss_d2
    out_ref[5] = pc_loss
    out_ref[6] = rgb_s
    out_ref[7] = dc


def kernel(rgb_pred, rgb_gt, depth_pred, depth_gt, t_list, X, Y, rgb_pc1,
           rgb_pc1_proj, valid_points, d1_proj, d2, d2_proj, d1):
    f32 = jnp.float32
    vmem = pl.BlockSpec(memory_space=pltpu.VMEM)
    hbm = pl.BlockSpec(memory_space=pltpu.MemorySpace.HBM)
    out = pl.pallas_call(
        _loss_body,
        out_shape=jax.ShapeDtypeStruct((8,), jnp.float32),
        in_specs=[vmem, vmem, vmem, vmem, vmem, vmem, vmem,
                  hbm, hbm, vmem, hbm, hbm, hbm, hbm],
        out_specs=pl.BlockSpec(memory_space=pltpu.SMEM),
        scratch_shapes=[
            pltpu.VMEM((3, 384, 384), f32), pltpu.VMEM((3, 384, 384), f32),
            pltpu.VMEM((512, 128), f32), pltpu.VMEM((512, 128), f32),
            pltpu.VMEM((512, 128), f32), pltpu.VMEM((512, 128), f32),
            pltpu.SemaphoreType.DMA((6,)),
        ],
    )(rgb_pred, rgb_gt, depth_pred, depth_gt, t_list, X, Y,
      rgb_pc1, rgb_pc1_proj, valid_points, d1_proj, d2, d2_proj, d1)
    return (out[0], out[1], out[2], out[3], out[4], out[5], out[6], out[7])


# speedup vs baseline: 1.2450x; 1.1811x over previous
import jax
import jax.numpy as jnp
from jax.experimental import pallas as pl
from jax.experimental.pallas import tpu as pltpu

_C1 = 0.01 ** 2
_C2 = 0.03 ** 2

_S = 3072
_BLK = 512
_DN = 192 * 256
_MED_K = (_DN - 1) // 2


def _box3(a):
    h_, w_ = a.shape
    left = jnp.concatenate([a[:, 1:2], a[:, : w_ - 1]], axis=1)
    right = jnp.concatenate([a[:, 1:w_], a[:, w_ - 2 : w_ - 1]], axis=1)
    h = left + a + right
    up = jnp.concatenate([h[1:2, :], h[: h_ - 1, :]], axis=0)
    dn = jnp.concatenate([h[1:h_, :], h[h_ - 2 : h_ - 1, :]], axis=0)
    return up + h + dn


def _sortable(x):
    b = jax.lax.bitcast_convert_type(x, jnp.int32)
    return jnp.where(b < 0, ~b, b ^ jnp.int32(-(2 ** 31)))


def _unsortable(prefix):
    fb = jnp.where(prefix < 0, prefix ^ jnp.int32(-(2 ** 31)), ~prefix)
    return jax.lax.bitcast_convert_type(fb, jnp.float32)


def _median_lower2(x, y):
    ux = _sortable(x)
    uy = _sortable(y)

    def body(i, carry):
        px, rkx, py, rky, done = carry
        bit = jnp.int32(31) - i
        bmask = jnp.left_shift(jnp.int32(1), bit)
        m = done | bmask
        cx = jnp.sum(((ux & m) == px).astype(jnp.int32))
        cy = jnp.sum(((uy & m) == py).astype(jnp.int32))
        tx = rkx >= cx
        ty = rky >= cy
        px = jnp.where(tx, px | bmask, px)
        rkx = jnp.where(tx, rkx - cx, rkx)
        py = jnp.where(ty, py | bmask, py)
        rky = jnp.where(ty, rky - cy, rky)
        return px, rkx, py, rky, done | bmask

    k = jnp.int32(_MED_K)
    px, _, py, _, _ = jax.lax.fori_loop(
        0, 32, body, (jnp.int32(0), k, jnp.int32(0), k, jnp.int32(0)))
    return _unsortable(px), _unsortable(py)


def _loss_body(rgbp_ref, rgbg_ref, dp_ref, dg_ref, t_ref, x_ref, y_ref,
               im1_hbm, im2_hbm, m_ref, d1p_hbm, d2_hbm, d2p_hbm, d1_hbm,
               out_ref,
               im1_ref, im2_ref, d1p_ref, d2_ref, d2p_ref, d1_ref, sems):
    f32 = jnp.float32

    copies = [
        pltpu.make_async_copy(im1_hbm.at[0], im1_ref, sems.at[0]),
        pltpu.make_async_copy(im2_hbm.at[0], im2_ref, sems.at[1]),
        pltpu.make_async_copy(d1p_hbm.reshape(512, 128), d1p_ref, sems.at[2]),
        pltpu.make_async_copy(d2_hbm.reshape(512, 128), d2_ref, sems.at[3]),
        pltpu.make_async_copy(d2p_hbm.reshape(512, 128), d2p_ref, sems.at[4]),
        pltpu.make_async_copy(d1_hbm.reshape(512, 128), d1_ref, sems.at[5]),
    ]
    for c in copies:
        c.start()

    ym = y_ref[0].T
    y0 = ym[0:1, :]
    y1 = ym[1:2, :]
    y2 = ym[2:3, :]

    bf = jnp.bfloat16
    yb0 = y0.astype(bf)
    yb1 = y1.astype(bf)
    yb2 = y2.astype(bf)

    def blk(i, carry):
        racc, colmin = carry
        xb = x_ref[0, pl.ds(i * _BLK, _BLK), :].astype(bf)
        e0 = xb[:, 0:1] - yb0
        e1 = xb[:, 1:2] - yb1
        e2 = xb[:, 2:3] - yb2
        dsq = e0 * e0 + e1 * e1 + e2 * e2
        rmin = jnp.min(dsq, axis=1, keepdims=True).astype(jnp.float32)
        racc = racc + jnp.sum(jnp.sqrt(jnp.maximum(rmin, 0.0)))
        colmin = jnp.minimum(colmin, jnp.min(dsq, axis=0, keepdims=True).astype(jnp.float32))
        return racc, colmin

    racc, colmin = jax.lax.fori_loop(
        0, _S // _BLK, blk,
        (f32(0.0), jnp.full((1, _S), jnp.inf, f32)))
    pc_loss = racc * f32(1.0 / _S) + jnp.sum(jnp.sqrt(colmin)) * f32(1.0 / _S)

    dr = rgbp_ref[0] - rgbg_ref[0]
    rgb_full = jnp.sum(dr * dr) * f32(1.0 / 2048.0)

    pred = dp_ref[...]
    gt = dg_ref[...]
    tp, tg = _median_lower2(pred, gt)
    sp = jnp.mean(jnp.abs(pred - tp))
    sg = jnp.mean(jnp.abs(gt - tg))
    pn = (pred - tp) / sp
    gn = (gt - tg) / sg
    depth_loss = jnp.mean((pn - gn) ** 2)

    t = t_ref[...]
    td = t[1:, :] - t[:-1, :]
    tn = jnp.sqrt(jnp.sum(td * td, axis=1, keepdims=True))
    loss_d1 = jnp.sum(tn) * f32(1.0 / 59.0)
    dd = tn[1:, :] - tn[:-1, :]
    loss_d2 = jnp.sum(dd * dd) * f32(1.0 / 58.0)

    copies[2].wait()
    copies[3].wait()
    copies[4].wait()
    copies[5].wait()
    dca = jnp.sum(jnp.abs(d1p_ref[...] - d2_ref[...])) * f32(1.0 / 65536.0)
    dcb = jnp.sum(jnp.abs(d2p_ref[...] - d1_ref[...])) * f32(1.0 / 65536.0)
    dc = 0.5 * dca + 0.5 * dcb

    copies[0].wait()
    copies[1].wait()
    m = m_ref[0, 0].astype(f32)
    msum = jnp.sum(m) * f32(3.0)
    acc = f32(0.0)
    for c in range(3):
        x = im1_ref[c]
        y = im2_ref[c]
        mu_x = _box3(x) * f32(1.0 / 9.0)
        mu_y = _box3(y) * f32(1.0 / 9.0)
        sxx = _box3(x * x) * f32(1.0 / 9.0) - mu_x * mu_x
        syy = _box3(y * y) * f32(1.0 / 9.0) - mu_y * mu_y
        sxy = _box3(x * y) * f32(1.0 / 9.0) - mu_x * mu_y
        n = (2.0 * mu_x * mu_y + _C1) * (2.0 * sxy + _C2)
        d = (mu_x * mu_x + mu_y * mu_y + _C1) * (sxx + syy + _C2)
        smap = jnp.clip((1.0 - n / d) * 0.5, 0.0, 1.0)
        diff = 0.15 * jnp.clip(jnp.abs(x - y), 0.0, 1.0) + 0.85 * smap
        acc = acc + jnp.sum(diff * m)
    rgb_s = acc / msum

    loss = (rgb_full + 0.04 * depth_loss + 0.1 * loss_d1 + 0.1 * loss_d2
            + pc_loss + rgb_s + dc)

    out_ref[0] = loss
    out_ref[1] = rgb_full
    out_ref[2] = depth_loss
    out_ref[3] = loss_d1
    out_ref[4] = loss_d2
    out_ref[5] = pc_loss
    out_ref[6] = rgb_s
    out_ref[7] = dc


def kernel(rgb_pred, rgb_gt, depth_pred, depth_gt, t_list, X, Y, rgb_pc1,
           rgb_pc1_proj, valid_points, d1_proj, d2, d2_proj, d1):
    f32 = jnp.float32
    vmem = pl.BlockSpec(memory_space=pltpu.VMEM)
    hbm = pl.BlockSpec(memory_space=pltpu.MemorySpace.HBM)
    out = pl.pallas_call(
        _loss_body,
        out_shape=jax.ShapeDtypeStruct((8,), jnp.float32),
        in_specs=[vmem, vmem, vmem, vmem, vmem, vmem, vmem,
                  hbm, hbm, vmem, hbm, hbm, hbm, hbm],
        out_specs=pl.BlockSpec(memory_space=pltpu.SMEM),
        scratch_shapes=[
            pltpu.VMEM((3, 384, 384), f32), pltpu.VMEM((3, 384, 384), f32),
            pltpu.VMEM((512, 128), f32), pltpu.VMEM((512, 128), f32),
            pltpu.VMEM((512, 128), f32), pltpu.VMEM((512, 128), f32),
            pltpu.SemaphoreType.DMA((6,)),
        ],
    )(rgb_pred, rgb_gt, depth_pred, depth_gt, t_list, X, Y,
      rgb_pc1, rgb_pc1_proj, valid_points, d1_proj, d2, d2_proj, d1)
    return (out[0], out[1], out[2], out[3], out[4], out[5], out[6], out[7])


# speedup vs baseline: 1.2775x; 1.0262x over previous
import jax
import jax.numpy as jnp
from jax.experimental import pallas as pl
from jax.experimental.pallas import tpu as pltpu

_C1 = 0.01 ** 2
_C2 = 0.03 ** 2

_S = 3072
_BLK = 1024
_DN = 192 * 256
_MED_K = (_DN - 1) // 2


def _box3(a):
    h_, w_ = a.shape
    left = jnp.concatenate([a[:, 1:2], a[:, : w_ - 1]], axis=1)
    right = jnp.concatenate([a[:, 1:w_], a[:, w_ - 2 : w_ - 1]], axis=1)
    h = left + a + right
    up = jnp.concatenate([h[1:2, :], h[: h_ - 1, :]], axis=0)
    dn = jnp.concatenate([h[1:h_, :], h[h_ - 2 : h_ - 1, :]], axis=0)
    return up + h + dn


def _sortable(x):
    b = jax.lax.bitcast_convert_type(x, jnp.int32)
    return jnp.where(b < 0, ~b, b ^ jnp.int32(-(2 ** 31)))


def _unsortable(prefix):
    fb = jnp.where(prefix < 0, prefix ^ jnp.int32(-(2 ** 31)), ~prefix)
    return jax.lax.bitcast_convert_type(fb, jnp.float32)


def _median_lower2(x, y):
    ux = _sortable(x)
    uy = _sortable(y)

    def body(i, carry):
        px, rkx, py, rky, done = carry
        bit = jnp.int32(31) - i
        bmask = jnp.left_shift(jnp.int32(1), bit)
        m = done | bmask
        cx = jnp.sum(((ux & m) == px).astype(jnp.int32))
        cy = jnp.sum(((uy & m) == py).astype(jnp.int32))
        tx = rkx >= cx
        ty = rky >= cy
        px = jnp.where(tx, px | bmask, px)
        rkx = jnp.where(tx, rkx - cx, rkx)
        py = jnp.where(ty, py | bmask, py)
        rky = jnp.where(ty, rky - cy, rky)
        return px, rkx, py, rky, done | bmask

    k = jnp.int32(_MED_K)
    px, _, py, _, _ = jax.lax.fori_loop(
        0, 32, body, (jnp.int32(0), k, jnp.int32(0), k, jnp.int32(0)))
    return _unsortable(px), _unsortable(py)


def _loss_body(rgbp_ref, rgbg_ref, dp_ref, dg_ref, t_ref, x_ref, y_ref,
               im1_hbm, im2_hbm, m_ref, d1p_hbm, d2_hbm, d2p_hbm, d1_hbm,
               out_ref,
               im1_ref, im2_ref, d1p_ref, d2_ref, d2p_ref, d1_ref, sems):
    f32 = jnp.float32

    copies = [
        pltpu.make_async_copy(im1_hbm.at[0], im1_ref, sems.at[0]),
        pltpu.make_async_copy(im2_hbm.at[0], im2_ref, sems.at[1]),
        pltpu.make_async_copy(d1p_hbm.reshape(512, 128), d1p_ref, sems.at[2]),
        pltpu.make_async_copy(d2_hbm.reshape(512, 128), d2_ref, sems.at[3]),
        pltpu.make_async_copy(d2p_hbm.reshape(512, 128), d2p_ref, sems.at[4]),
        pltpu.make_async_copy(d1_hbm.reshape(512, 128), d1_ref, sems.at[5]),
    ]
    for c in copies:
        c.start()

    ym = y_ref[0].T
    y0 = ym[0:1, :]
    y1 = ym[1:2, :]
    y2 = ym[2:3, :]

    bf = jnp.bfloat16
    yb0 = y0.astype(bf)
    yb1 = y1.astype(bf)
    yb2 = y2.astype(bf)

    def blk(i, carry):
        racc, colmin = carry
        xb = x_ref[0, pl.ds(i * _BLK, _BLK), :].astype(bf)
        e0 = xb[:, 0:1] - yb0
        e1 = xb[:, 1:2] - yb1
        e2 = xb[:, 2:3] - yb2
        dsq = e0 * e0 + e1 * e1 + e2 * e2
        rmin = jnp.min(dsq, axis=1, keepdims=True).astype(jnp.float32)
        racc = racc + jnp.sum(jnp.sqrt(jnp.maximum(rmin, 0.0)))
        colmin = jnp.minimum(colmin, jnp.min(dsq, axis=0, keepdims=True).astype(jnp.float32))
        return racc, colmin

    racc, colmin = jax.lax.fori_loop(
        0, _S // _BLK, blk,
        (f32(0.0), jnp.full((1, _S), jnp.inf, f32)))
    pc_loss = racc * f32(1.0 / _S) + jnp.sum(jnp.sqrt(colmin)) * f32(1.0 / _S)

    dr = rgbp_ref[0] - rgbg_ref[0]
    rgb_full = jnp.sum(dr * dr) * f32(1.0 / 2048.0)

    pred = dp_ref[...]
    gt = dg_ref[...]
    tp, tg = _median_lower2(pred, gt)
    sp = jnp.mean(jnp.abs(pred - tp))
    sg = jnp.mean(jnp.abs(gt - tg))
    pn = (pred - tp) / sp
    gn = (gt - tg) / sg
    depth_loss = jnp.mean((pn - gn) ** 2)

    t = t_ref[...]
    td = t[1:, :] - t[:-1, :]
    tn = jnp.sqrt(jnp.sum(td * td, axis=1, keepdims=True))
    loss_d1 = jnp.sum(tn) * f32(1.0 / 59.0)
    dd = tn[1:, :] - tn[:-1, :]
    loss_d2 = jnp.sum(dd * dd) * f32(1.0 / 58.0)

    copies[2].wait()
    copies[3].wait()
    copies[4].wait()
    copies[5].wait()
    dca = jnp.sum(jnp.abs(d1p_ref[...] - d2_ref[...])) * f32(1.0 / 65536.0)
    dcb = jnp.sum(jnp.abs(d2p_ref[...] - d1_ref[...])) * f32(1.0 / 65536.0)
    dc = 0.5 * dca + 0.5 * dcb

    copies[0].wait()
    copies[1].wait()
    m = m_ref[0, 0].astype(f32)
    msum = jnp.sum(m) * f32(3.0)
    acc = f32(0.0)
    for c in range(3):
        x = im1_ref[c]
        y = im2_ref[c]
        mu_x = _box3(x) * f32(1.0 / 9.0)
        mu_y = _box3(y) * f32(1.0 / 9.0)
        sxx = _box3(x * x) * f32(1.0 / 9.0) - mu_x * mu_x
        syy = _box3(y * y) * f32(1.0 / 9.0) - mu_y * mu_y
        sxy = _box3(x * y) * f32(1.0 / 9.0) - mu_x * mu_y
        n = (2.0 * mu_x * mu_y + _C1) * (2.0 * sxy + _C2)
        d = (mu_x * mu_x + mu_y * mu_y + _C1) * (sxx + syy + _C2)
        smap = jnp.clip((1.0 - n / d) * 0.5, 0.0, 1.0)
        diff = 0.15 * jnp.clip(jnp.abs(x - y), 0.0, 1.0) + 0.85 * smap
        acc = acc + jnp.sum(diff * m)
    rgb_s = acc / msum

    loss = (rgb_full + 0.04 * depth_loss + 0.1 * loss_d1 + 0.1 * loss_d2
            + pc_loss + rgb_s + dc)

    out_ref[0] = loss
    out_ref[1] = rgb_full
    out_ref[2] = depth_loss
    out_ref[3] = loss_d1
    out_ref[4] = loss_d2
    out_ref[5] = pc_loss
    out_ref[6] = rgb_s
    out_ref[7] = dc


def kernel(rgb_pred, rgb_gt, depth_pred, depth_gt, t_list, X, Y, rgb_pc1,
           rgb_pc1_proj, valid_points, d1_proj, d2, d2_proj, d1):
    f32 = jnp.float32
    vmem = pl.BlockSpec(memory_space=pltpu.VMEM)
    hbm = pl.BlockSpec(memory_space=pltpu.MemorySpace.HBM)
    out = pl.pallas_call(
        _loss_body,
        out_shape=jax.ShapeDtypeStruct((8,), jnp.float32),
        in_specs=[vmem, vmem, vmem, vmem, vmem, vmem, vmem,
                  hbm, hbm, vmem, hbm, hbm, hbm, hbm],
        out_specs=pl.BlockSpec(memory_space=pltpu.SMEM),
        scratch_shapes=[
            pltpu.VMEM((3, 384, 384), f32), pltpu.VMEM((3, 384, 384), f32),
            pltpu.VMEM((512, 128), f32), pltpu.VMEM((512, 128), f32),
            pltpu.VMEM((512, 128), f32), pltpu.VMEM((512, 128), f32),
            pltpu.SemaphoreType.DMA((6,)),
        ],
    )(rgb_pred, rgb_gt, depth_pred, depth_gt, t_list, X, Y,
      rgb_pc1, rgb_pc1_proj, valid_points, d1_proj, d2, d2_proj, d1)
    return (out[0], out[1], out[2], out[3], out[4], out[5], out[6], out[7])
